# baseline (speedup 1.0000x reference)
import jax
import jax.numpy as jnp
from jax.experimental import pallas as pl
from jax.experimental.pallas import tpu as pltpu

B = 512
D = 4096
E = 64
K = 8
IB = 16


def _fused_kernel(x_ref, wt_ref, out_ref, mask_ref, ones_ref,
                  gatest_s, maskt_s):
    i = pl.program_id(0)

    @pl.when(i == 0)
    def _router():
        x = x_ref[...]
        w = wt_ref[...]
        lt = jax.lax.dot_general(w, x, (((1,), (1,)), ((), ())),
                                 preferred_element_type=jnp.float32)
        m = jnp.max(lt, axis=0, keepdims=True)
        ex = jnp.exp(lt - m)
        s = jnp.sum(ex, axis=0, keepdims=True)
        gt = ex / s

        row = jax.lax.broadcasted_iota(jnp.int32, (E, B), 0)
        work = gt
        mask_t = jnp.zeros((E, B), jnp.float32)
        for _ in range(K):
            mx = jnp.max(work, axis=0, keepdims=True)
            cand = jnp.where(work == mx, row, E)
            first = jnp.min(cand, axis=0, keepdims=True)
            pick = row == first
            mask_t = jnp.where(pick, 1.0, mask_t)
            work = jnp.where(pick, -1.0, work)

        gatest_s[...] = jnp.transpose(gt)
        maskt_s[...] = mask_t
        mask_ref[...] = jnp.transpose(mask_t)
        ones_ref[...] = jnp.ones((B, E), jnp.float32)

    mt = maskt_s[...]
    g_blk = gatest_s[pl.ds(i * IB, IB), :]
    out_ref[...] = g_blk[:, :, None] * mt[None, :, :]


def kernel(x, W):
    outt, mask, ones = pl.pallas_call(
        _fused_kernel,
        grid=(B // IB,),
        in_specs=[
            pl.BlockSpec((B, D), lambda i: (0, 0)),
            pl.BlockSpec((E, D), lambda i: (0, 0)),
        ],
        out_specs=(
            pl.BlockSpec((IB, E, B), lambda i: (i, 0, 0)),
            pl.BlockSpec((B, E), lambda i: (0, 0)),
            pl.BlockSpec((B, E), lambda i: (0, 0)),
        ),
        out_shape=(
            jax.ShapeDtypeStruct((B, E, B), jnp.float32),
            jax.ShapeDtypeStruct((B, E), jnp.float32),
            jax.ShapeDtypeStruct((B, E), jnp.float32),
        ),
        scratch_shapes=[
            pltpu.VMEM((B, E), jnp.float32),
            pltpu.VMEM((E, B), jnp.float32),
        ],
    )(x, W)
    combine = jnp.transpose(outt, (0, 2, 1))
    return (combine, mask, ones)


# speedup vs baseline: 1.0824x; 1.0824x over previous
import jax
import jax.numpy as jnp
from jax.experimental import pallas as pl
from jax.experimental.pallas import tpu as pltpu

B = 512
D = 4096
E = 64
K = 8
IB = 32


XC = 128
NXC = B // XC


def _fused_kernel(x_hbm, wt_ref, out_ref, mask_ref, ones_ref,
                  gatest_s, maskt_s, xbuf, sem):
    i = pl.program_id(0)

    @pl.when(i == 0)
    def _router():
        w = wt_ref[...]

        def chunk_copy(c, slot):
            return pltpu.make_async_copy(
                x_hbm.at[pl.ds(c * XC, XC)], xbuf.at[slot], sem.at[slot])

        chunk_copy(0, 0).start()
        row = jax.lax.broadcasted_iota(jnp.int32, (E, XC), 0)
        for c in range(NXC):
            slot = c % 2
            if c + 1 < NXC:
                chunk_copy(c + 1, 1 - slot).start()
            chunk_copy(c, slot).wait()
            lt = jax.lax.dot_general(w, xbuf[slot], (((1,), (1,)), ((), ())),
                                     preferred_element_type=jnp.float32)
            m = jnp.max(lt, axis=0, keepdims=True)
            ex = jnp.exp(lt - m)
            s = jnp.sum(ex, axis=0, keepdims=True)
            gt = ex / s

            work = gt
            mask_t = jnp.zeros((E, XC), jnp.float32)
            for _ in range(K):
                mx = jnp.max(work, axis=0, keepdims=True)
                cand = jnp.where(work == mx, row, E)
                first = jnp.min(cand, axis=0, keepdims=True)
                pick = row == first
                mask_t = jnp.where(pick, 1.0, mask_t)
                work = jnp.where(pick, -1.0, work)

            cols = pl.ds(c * XC, XC)
            gatest_s[cols, :] = jnp.transpose(gt)
            maskt_s[:, cols] = mask_t
            mask_ref[cols, :] = jnp.transpose(mask_t)
        ones_ref[...] = jnp.ones((B, E), jnp.float32)

    mt = maskt_s[...]
    g_blk = gatest_s[pl.ds(i * IB, IB), :]
    out_ref[...] = g_blk[:, :, None] * mt[None, :, :]


def kernel(x, W):
    outt, mask, ones = pl.pallas_call(
        _fused_kernel,
        grid=(B // IB,),
        in_specs=[
            pl.BlockSpec(memory_space=pl.ANY),
            pl.BlockSpec((E, D), lambda i: (0, 0)),
        ],
        out_specs=(
            pl.BlockSpec((IB, E, B), lambda i: (i, 0, 0)),
            pl.BlockSpec((B, E), lambda i: (0, 0)),
            pl.BlockSpec((B, E), lambda i: (0, 0)),
        ),
        out_shape=(
            jax.ShapeDtypeStruct((B, E, B), jnp.float32),
            jax.ShapeDtypeStruct((B, E), jnp.float32),
            jax.ShapeDtypeStruct((B, E), jnp.float32),
        ),
        scratch_shapes=[
            pltpu.VMEM((B, E), jnp.float32),
            pltpu.VMEM((E, B), jnp.float32),
            pltpu.VMEM((2, XC, D), jnp.float32),
            pltpu.SemaphoreType.DMA((2,)),
        ],
    )(x, W)
    combine = jnp.transpose(outt, (0, 2, 1))
    return (combine, mask, ones)


# speedup vs baseline: 1.1376x; 1.0510x over previous
import jax
import jax.numpy as jnp
from jax.experimental import pallas as pl
from jax.experimental.pallas import tpu as pltpu

B = 512
D = 4096
E = 64
K = 8
IB = 32


def _fused_kernel(x_ref, wt_ref, out_ref, mask_ref, ones_ref,
                  gatest_s, maskt_s):
    i = pl.program_id(0)

    @pl.when(i == 0)
    def _router():
        x = x_ref[...]
        w = wt_ref[...]
        lt = jax.lax.dot_general(w, x, (((1,), (1,)), ((), ())),
                                 preferred_element_type=jnp.float32)
        m = jnp.max(lt, axis=0, keepdims=True)
        ex = jnp.exp(lt - m)
        s = jnp.sum(ex, axis=0, keepdims=True)
        gt = ex / s

        row = jax.lax.broadcasted_iota(jnp.int32, (E, B), 0)
        work = gt
        mask_t = jnp.zeros((E, B), jnp.float32)
        for _ in range(K):
            mx = jnp.max(work, axis=0, keepdims=True)
            cand = jnp.where(work == mx, row, E)
            first = jnp.min(cand, axis=0, keepdims=True)
            pick = row == first
            mask_t = jnp.where(pick, 1.0, mask_t)
            work = jnp.where(pick, -1.0, work)

        gatest_s[...] = jnp.transpose(gt)
        maskt_s[...] = mask_t
        mask_ref[...] = jnp.transpose(mask_t)
        ones_ref[...] = jnp.ones((B, E), jnp.float32)

    mt = maskt_s[...]
    g_blk = gatest_s[pl.ds(i * IB, IB), :]
    out_ref[...] = g_blk[:, :, None] * mt[None, :, :]


def kernel(x, W):
    outt, mask, ones = pl.pallas_call(
        _fused_kernel,
        grid=(B // IB,),
        in_specs=[
            pl.BlockSpec((B, D), lambda i: (0, 0)),
            pl.BlockSpec((E, D), lambda i: (0, 0)),
        ],
        out_specs=(
            pl.BlockSpec((IB, E, B), lambda i: (i, 0, 0)),
            pl.BlockSpec((B, E), lambda i: (0, 0)),
            pl.BlockSpec((B, E), lambda i: (0, 0)),
        ),
        out_shape=(
            jax.ShapeDtypeStruct((B, E, B), jnp.float32),
            jax.ShapeDtypeStruct((B, E), jnp.float32),
            jax.ShapeDtypeStruct((B, E), jnp.float32),
        ),
        scratch_shapes=[
            pltpu.VMEM((B, E), jnp.float32),
            pltpu.VMEM((E, B), jnp.float32),
        ],
    )(x, W)
    combine = jnp.transpose(outt, (0, 2, 1))
    return (combine, mask, ones)
